# baseline (speedup 1.0000x reference)
import functools

import jax
import jax.numpy as jnp
from jax import lax
from jax.experimental import pallas as pl
from jax.experimental.pallas import tpu as pltpu

NX = 64
BATCH = 16384


def _body(ids_ref, table_ref, out_ref, sem):
    def issue(i, carry):
        r = ids_ref[i]
        pltpu.make_async_copy(table_ref.at[r], out_ref.at[i], sem).start()
        return carry

    lax.fori_loop(0, BATCH, issue, 0)

    def drain(i, carry):
        pltpu.make_async_copy(table_ref.at[0], out_ref.at[i], sem).wait()
        return carry

    lax.fori_loop(0, BATCH, drain, 0)


@jax.jit
def _tc_gather(ids, start_state):
    return pl.pallas_call(
        _body,
        out_shape=jax.ShapeDtypeStruct((BATCH, NX), jnp.float32),
        in_specs=[
            pl.BlockSpec(memory_space=pltpu.SMEM),
            pl.BlockSpec(memory_space=pl.ANY),
        ],
        out_specs=pl.BlockSpec(memory_space=pl.ANY),
        scratch_shapes=[pltpu.SemaphoreType.DMA],
    )(ids, start_state)


def kernel(ids, start_state):
    return _tc_gather(ids.astype(jnp.int32), start_state)


# speedup vs baseline: 1.4799x; 1.4799x over previous
import functools

import jax
import jax.numpy as jnp
from jax import lax
from jax.experimental import pallas as pl
from jax.experimental.pallas import tpu as pltpu
from jax.experimental.pallas import tpu_sc as plsc

NX = 64
BATCH = 16384
NUM_CORES = 2
NUM_SUBCORES = 16
NUM_WORKERS = NUM_CORES * NUM_SUBCORES

SC_ROWS = 10240
TC_ROWS = BATCH - SC_ROWS
SC_B = SC_ROWS // NUM_WORKERS
TC_SEMS = 4
TC_Q = TC_ROWS // TC_SEMS


@functools.partial(
    pl.kernel,
    out_type=jax.ShapeDtypeStruct((SC_ROWS, NX), jnp.float32),
    mesh=plsc.VectorSubcoreMesh(core_axis_name="c", subcore_axis_name="s"),
    scratch_types=[
        pltpu.VMEM((SC_B,), jnp.int32),
        pltpu.VMEM((SC_B, NX), jnp.float32),
        pltpu.SemaphoreType.DMA,
    ],
    compiler_params=pltpu.CompilerParams(use_tc_tiling_on_sc=True),
)
def _sc_gather(ids_hbm, table_hbm, out_hbm, ids_v, rows_v, sem):
    wid = lax.axis_index("s") * NUM_CORES + lax.axis_index("c")
    base = wid * SC_B
    pltpu.sync_copy(ids_hbm.at[wid], ids_v)

    def issue(s, carry):
        vec = ids_v[pl.ds(s * 16, 16)]
        for l in range(16):
            r = vec[l]
            pltpu.make_async_copy(
                table_hbm.at[r], rows_v.at[s * 16 + l], sem
            ).start()
        return carry

    lax.fori_loop(0, SC_B // 16, issue, 0)

    def drain(j, carry):
        pltpu.make_async_copy(table_hbm.at[0], rows_v.at[j], sem).wait()
        return carry

    lax.fori_loop(0, SC_B, drain, 0)

    pltpu.sync_copy(rows_v, out_hbm.at[pl.ds(base, SC_B)])


def _tc_body(ids_ref, table_ref, out_ref, *sems):
    for s in range(TC_SEMS):
        def issue(i, carry, s=s):
            r = ids_ref[s * TC_Q + i]
            pltpu.make_async_copy(
                table_ref.at[r], out_ref.at[s * TC_Q + i], sems[s]
            ).start()
            return carry

        lax.fori_loop(0, TC_Q, issue, 0, unroll=8)
    for s in range(TC_SEMS):
        pltpu.make_async_copy(
            table_ref.at[pl.ds(0, TC_Q)],
            out_ref.at[pl.ds(s * TC_Q, TC_Q)],
            sems[s],
        ).wait()


def _tc_gather(ids, start_state):
    return pl.pallas_call(
        _tc_body,
        out_shape=jax.ShapeDtypeStruct((TC_ROWS, NX), jnp.float32),
        in_specs=[
            pl.BlockSpec(memory_space=pltpu.SMEM),
            pl.BlockSpec(memory_space=pl.ANY),
        ],
        out_specs=pl.BlockSpec(memory_space=pl.ANY),
        scratch_shapes=[pltpu.SemaphoreType.DMA] * TC_SEMS,
    )(ids, start_state)


def kernel(ids, start_state):
    ids32 = ids.astype(jnp.int32)
    sc_out = _sc_gather(
        ids32[:SC_ROWS].reshape(NUM_WORKERS, SC_B), start_state
    )
    tc_out = _tc_gather(ids32[SC_ROWS:], start_state)
    return jnp.concatenate([sc_out, tc_out], axis=0)
